# Initial kernel scaffold; baseline (speedup 1.0000x reference)
import jax
import jax.numpy as jnp
from jax.experimental import pallas as pl


def kernel(x, edge_index, edge_attr, candidate_idxs, batch, lin_in_w, lin_in_b, w1, b1, w2, b2, g1, be1, g2, be2, lin_out_w, lin_out_b):
    raise NotImplementedError("write your pallas kernel here")


# speedup vs baseline: 1.7144x; 1.7144x over previous
import functools

import jax
import jax.numpy as jnp
from jax import lax
from jax.experimental import pallas as pl
from jax.experimental.pallas import tpu as pltpu
from jax.experimental.pallas import tpu_sc as plsc

N = 50000
E = 800000
C = 5000
G = 500
EMB = 32
ED = 4
IN_DIM = 2
NLAYERS = 4
EPS = 1e-5

NC = 2
NS = 16
NW = NC * NS

ECHUNK = 128
CHUNKS_PER_W = 196
E_PAD = NW * CHUNKS_PER_W * ECHUNK

NACC = 51200
ROWS_PER_SUB = NACC // NS

C_CHUNK = 80
C_PAD = NW * 2 * C_CHUNK

ETILE = 2048
EGRID = E_PAD // ETILE
NTILE = 2000
NGRID = N // NTILE

_MESH = plsc.VectorSubcoreMesh(core_axis_name="c", subcore_axis_name="s")
_SC_PARAMS = pltpu.CompilerParams(use_tc_tiling_on_sc=False,
                                  needs_layout_passes=False)


@functools.partial(
    pl.kernel,
    out_type=(
        jax.ShapeDtypeStruct((E_PAD, EMB), jnp.float32),
        jax.ShapeDtypeStruct((E_PAD, EMB), jnp.float32),
    ),
    mesh=_MESH,
    compiler_params=_SC_PARAMS,
    scratch_types=[
        pltpu.VMEM((ECHUNK,), jnp.int32),
        pltpu.VMEM((ECHUNK,), jnp.int32),
        pltpu.VMEM((ECHUNK, EMB), jnp.float32),
        pltpu.VMEM((ECHUNK, EMB), jnp.float32),
        pltpu.SemaphoreType.DMA,
        pltpu.SemaphoreType.DMA,
    ],
)
def _gather_pq(p_hbm, q_hbm, dst_hbm, src_hbm, g1_hbm, g2_hbm,
               idx1, idx2, buf1, buf2, sem1, sem2):
    wid = lax.axis_index("s") * NC + lax.axis_index("c")

    def chunk(ci, carry):
        base = (wid * CHUNKS_PER_W + ci) * ECHUNK
        pltpu.sync_copy(dst_hbm.at[pl.ds(base, ECHUNK)], idx1)
        pltpu.sync_copy(src_hbm.at[pl.ds(base, ECHUNK)], idx2)
        cp1 = pltpu.async_copy(p_hbm.at[idx1], buf1, sem1)
        cp2 = pltpu.async_copy(q_hbm.at[idx2], buf2, sem2)
        cp1.wait()
        cp2.wait()
        pltpu.sync_copy(buf1, g1_hbm.at[pl.ds(base, ECHUNK)])
        pltpu.sync_copy(buf2, g2_hbm.at[pl.ds(base, ECHUNK)])
        return carry

    lax.fori_loop(0, CHUNKS_PER_W, chunk, 0)


@functools.partial(
    pl.kernel,
    out_type=jax.ShapeDtypeStruct((NC, NACC, EMB), jnp.float32),
    mesh=_MESH,
    compiler_params=_SC_PARAMS,
    scratch_types=[
        pltpu.VMEM((ECHUNK, EMB), jnp.float32),
        pltpu.VMEM((ECHUNK,), jnp.int32),
        pltpu.VMEM((ECHUNK, EMB), jnp.float32),
        pltpu.VMEM((EMB,), jnp.float32),
        pltpu.VMEM((EMB,), jnp.float32),
        pltpu.VMEM_SHARED((NACC, EMB), jnp.float32),
    ],
)
def _scatter_layer(m2_hbm, dst_hbm, s_hbm, t_hbm, out_hbm,
                   mbuf, idxv, zbuf, s2v, t2v, acc):
    cid = lax.axis_index("c")
    sid = lax.axis_index("s")
    wid = sid * NC + cid

    zero = jnp.zeros((16,), jnp.float32)

    def zrow(r, carry):
        zbuf[r, pl.ds(0, 16)] = zero
        zbuf[r, pl.ds(16, 16)] = zero
        return carry

    lax.fori_loop(0, ECHUNK, zrow, 0)

    def zstripe(j, carry):
        pltpu.sync_copy(zbuf, acc.at[pl.ds(sid * ROWS_PER_SUB + j * ECHUNK, ECHUNK)])
        return carry

    lax.fori_loop(0, ROWS_PER_SUB // ECHUNK, zstripe, 0)

    pltpu.sync_copy(s_hbm, s2v)
    pltpu.sync_copy(t_hbm, t2v)
    sa = s2v[pl.ds(0, 16)]
    sb = s2v[pl.ds(16, 16)]
    ta = t2v[pl.ds(0, 16)]
    tb = t2v[pl.ds(16, 16)]

    plsc.subcore_barrier()

    def chunk(ci, carry):
        base = (wid * CHUNKS_PER_W + ci) * ECHUNK
        pltpu.sync_copy(m2_hbm.at[pl.ds(base, ECHUNK)], mbuf)
        pltpu.sync_copy(dst_hbm.at[pl.ds(base, ECHUNK)], idxv)

        def row(r, rc):
            mbuf[r, pl.ds(0, 16)] = jnp.maximum(
                mbuf[r, pl.ds(0, 16)] * sa + ta, 0.0)
            mbuf[r, pl.ds(16, 16)] = jnp.maximum(
                mbuf[r, pl.ds(16, 16)] * sb + tb, 0.0)
            return rc

        lax.fori_loop(0, ECHUNK, row, 0)
        pltpu.sync_copy(mbuf, acc.at[idxv], add=True)
        return carry

    lax.fori_loop(0, CHUNKS_PER_W, chunk, 0)

    plsc.subcore_barrier()
    pltpu.sync_copy(acc.at[pl.ds(sid * ROWS_PER_SUB, ROWS_PER_SUB)],
                    out_hbm.at[cid, pl.ds(sid * ROWS_PER_SUB, ROWS_PER_SUB)])


@functools.partial(
    pl.kernel,
    out_type=(
        jax.ShapeDtypeStruct((C_PAD, EMB), jnp.float32),
        jax.ShapeDtypeStruct((C_PAD,), jnp.int32),
    ),
    mesh=_MESH,
    compiler_params=_SC_PARAMS,
    scratch_types=[
        pltpu.VMEM((C_CHUNK,), jnp.int32),
        pltpu.VMEM((C_CHUNK, EMB), jnp.float32),
        pltpu.VMEM((C_CHUNK,), jnp.int32),
        pltpu.VMEM((N,), jnp.int32),
        pltpu.SemaphoreType.DMA,
    ],
)
def _readout_gather(h_hbm, cand_hbm, batch_hbm, hc_hbm, seg_hbm,
                    idxc, bufh, segbuf, batchv, sem):
    wid = lax.axis_index("s") * NC + lax.axis_index("c")
    pltpu.sync_copy(batch_hbm, batchv)
    for k in range(2):
        base = wid * (2 * C_CHUNK) + k * C_CHUNK
        pltpu.sync_copy(cand_hbm.at[pl.ds(base, C_CHUNK)], idxc)
        pltpu.async_copy(h_hbm.at[idxc], bufh, sem).wait()
        for t in range(C_CHUNK // 16):
            iv = idxc[pl.ds(t * 16, 16)]
            segbuf[pl.ds(t * 16, 16)] = plsc.load_gather(batchv, [iv])
        pltpu.sync_copy(bufh, hc_hbm.at[pl.ds(base, C_CHUNK)])
        pltpu.sync_copy(segbuf, seg_hbm.at[pl.ds(base, C_CHUNK)])


def _node_first_body(x_ref, w_ref, b_ref, at_ref, bt_ref, h_ref, p_ref, q_ref):
    h = jnp.dot(x_ref[...], w_ref[...], preferred_element_type=jnp.float32) + b_ref[...]
    h_ref[...] = h
    p_ref[...] = jnp.dot(h, at_ref[...], preferred_element_type=jnp.float32)
    q_ref[...] = jnp.dot(h, bt_ref[...], preferred_element_type=jnp.float32)


def _node_first(x, w_in_t, b_in, a_t, b_t):
    return pl.pallas_call(
        _node_first_body,
        grid=(NGRID,),
        in_specs=[
            pl.BlockSpec((NTILE, IN_DIM), lambda i: (i, 0)),
            pl.BlockSpec((IN_DIM, EMB), lambda i: (0, 0)),
            pl.BlockSpec((1, EMB), lambda i: (0, 0)),
            pl.BlockSpec((EMB, EMB), lambda i: (0, 0)),
            pl.BlockSpec((EMB, EMB), lambda i: (0, 0)),
        ],
        out_specs=[
            pl.BlockSpec((NTILE, EMB), lambda i: (i, 0)),
            pl.BlockSpec((NTILE, EMB), lambda i: (i, 0)),
            pl.BlockSpec((NTILE, EMB), lambda i: (i, 0)),
        ],
        out_shape=[
            jax.ShapeDtypeStruct((N, EMB), jnp.float32),
            jax.ShapeDtypeStruct((N, EMB), jnp.float32),
            jax.ShapeDtypeStruct((N, EMB), jnp.float32),
        ],
    )(x, w_in_t, b_in, a_t, b_t)


def _node_update_body(h_ref, a0_ref, a1_ref, at_ref, bt_ref,
                      ho_ref, p_ref, q_ref):
    h = h_ref[...] + a0_ref[...] + a1_ref[...]
    ho_ref[...] = h
    p_ref[...] = jnp.dot(h, at_ref[...], preferred_element_type=jnp.float32)
    q_ref[...] = jnp.dot(h, bt_ref[...], preferred_element_type=jnp.float32)


def _node_update(h, a0, a1, a_t, b_t):
    return pl.pallas_call(
        _node_update_body,
        grid=(NGRID,),
        in_specs=[
            pl.BlockSpec((NTILE, EMB), lambda i: (i, 0)),
            pl.BlockSpec((NTILE, EMB), lambda i: (i, 0)),
            pl.BlockSpec((NTILE, EMB), lambda i: (i, 0)),
            pl.BlockSpec((EMB, EMB), lambda i: (0, 0)),
            pl.BlockSpec((EMB, EMB), lambda i: (0, 0)),
        ],
        out_specs=[
            pl.BlockSpec((NTILE, EMB), lambda i: (i, 0)),
            pl.BlockSpec((NTILE, EMB), lambda i: (i, 0)),
            pl.BlockSpec((NTILE, EMB), lambda i: (i, 0)),
        ],
        out_shape=[
            jax.ShapeDtypeStruct((N, EMB), jnp.float32),
            jax.ShapeDtypeStruct((N, EMB), jnp.float32),
            jax.ShapeDtypeStruct((N, EMB), jnp.float32),
        ],
    )(h, a0, a1, a_t, b_t)


def _node_final_body(h_ref, a0_ref, a1_ref, ho_ref):
    ho_ref[...] = h_ref[...] + a0_ref[...] + a1_ref[...]


def _node_final(h, a0, a1):
    return pl.pallas_call(
        _node_final_body,
        grid=(NGRID,),
        in_specs=[
            pl.BlockSpec((NTILE, EMB), lambda i: (i, 0)),
            pl.BlockSpec((NTILE, EMB), lambda i: (i, 0)),
            pl.BlockSpec((NTILE, EMB), lambda i: (i, 0)),
        ],
        out_specs=pl.BlockSpec((NTILE, EMB), lambda i: (i, 0)),
        out_shape=jax.ShapeDtypeStruct((N, EMB), jnp.float32),
    )(h, a0, a1)


def _edge1_body(g1_ref, g2_ref, ea_ref, ct_ref, b1_ref, m_ref, s1_ref, s2_ref):
    i = pl.program_id(0)
    m = (g1_ref[...] + g2_ref[...]
         + jnp.dot(ea_ref[...], ct_ref[...], preferred_element_type=jnp.float32)
         + b1_ref[...])
    m_ref[...] = m
    rows = lax.broadcasted_iota(jnp.int32, (ETILE, EMB), 0) + i * ETILE
    mask = rows < E
    p1 = jnp.sum(jnp.where(mask, m, 0.0), axis=0)
    p2 = jnp.sum(jnp.where(mask, m * m, 0.0), axis=0)
    rowsel = lax.broadcasted_iota(jnp.int32, (8, EMB), 0) == 0
    c1 = jnp.where(rowsel, p1[None, :], 0.0)
    c2 = jnp.where(rowsel, p2[None, :], 0.0)

    @pl.when(i == 0)
    def _():
        s1_ref[...] = c1
        s2_ref[...] = c2

    @pl.when(i > 0)
    def _():
        s1_ref[...] += c1
        s2_ref[...] += c2


def _edge_pass1(gd, gs, ea, c_t, b1r):
    return pl.pallas_call(
        _edge1_body,
        grid=(EGRID,),
        in_specs=[
            pl.BlockSpec((ETILE, EMB), lambda i: (i, 0)),
            pl.BlockSpec((ETILE, EMB), lambda i: (i, 0)),
            pl.BlockSpec((ETILE, ED), lambda i: (i, 0)),
            pl.BlockSpec((ED, EMB), lambda i: (0, 0)),
            pl.BlockSpec((1, EMB), lambda i: (0, 0)),
        ],
        out_specs=[
            pl.BlockSpec((ETILE, EMB), lambda i: (i, 0)),
            pl.BlockSpec((8, EMB), lambda i: (0, 0)),
            pl.BlockSpec((8, EMB), lambda i: (0, 0)),
        ],
        out_shape=[
            jax.ShapeDtypeStruct((E_PAD, EMB), jnp.float32),
            jax.ShapeDtypeStruct((8, EMB), jnp.float32),
            jax.ShapeDtypeStruct((8, EMB), jnp.float32),
        ],
    )(gd, gs, ea, c_t, b1r)


def _edge2_body(m_ref, s1_ref, t1_ref, w2t_ref, b2_ref, m2_ref, u1_ref, u2_ref):
    i = pl.program_id(0)
    a = jnp.maximum(m_ref[...] * s1_ref[...] + t1_ref[...], 0.0)
    m2 = jnp.dot(a, w2t_ref[...], preferred_element_type=jnp.float32) + b2_ref[...]
    m2_ref[...] = m2
    rows = lax.broadcasted_iota(jnp.int32, (ETILE, EMB), 0) + i * ETILE
    mask = rows < E
    p1 = jnp.sum(jnp.where(mask, m2, 0.0), axis=0)
    p2 = jnp.sum(jnp.where(mask, m2 * m2, 0.0), axis=0)
    rowsel = lax.broadcasted_iota(jnp.int32, (8, EMB), 0) == 0
    c1 = jnp.where(rowsel, p1[None, :], 0.0)
    c2 = jnp.where(rowsel, p2[None, :], 0.0)

    @pl.when(i == 0)
    def _():
        u1_ref[...] = c1
        u2_ref[...] = c2

    @pl.when(i > 0)
    def _():
        u1_ref[...] += c1
        u2_ref[...] += c2


def _edge_pass2(m, s1r, t1r, w2t, b2r):
    return pl.pallas_call(
        _edge2_body,
        grid=(EGRID,),
        in_specs=[
            pl.BlockSpec((ETILE, EMB), lambda i: (i, 0)),
            pl.BlockSpec((1, EMB), lambda i: (0, 0)),
            pl.BlockSpec((1, EMB), lambda i: (0, 0)),
            pl.BlockSpec((EMB, EMB), lambda i: (0, 0)),
            pl.BlockSpec((1, EMB), lambda i: (0, 0)),
        ],
        out_specs=[
            pl.BlockSpec((ETILE, EMB), lambda i: (i, 0)),
            pl.BlockSpec((8, EMB), lambda i: (0, 0)),
            pl.BlockSpec((8, EMB), lambda i: (0, 0)),
        ],
        out_shape=[
            jax.ShapeDtypeStruct((E_PAD, EMB), jnp.float32),
            jax.ShapeDtypeStruct((8, EMB), jnp.float32),
            jax.ShapeDtypeStruct((8, EMB), jnp.float32),
        ],
    )(m, s1r, t1r, w2t, b2r)


def _readout_body(hc_ref, seg_ref, wout_ref, bout_ref, out_ref):
    hc = hc_ref[...]
    logits = jnp.sum(hc * wout_ref[...], axis=1, keepdims=True) + bout_ref[...]
    seg = seg_ref[...]
    gids = lax.broadcasted_iota(jnp.int32, (C_PAD, G), 1)
    valid = lax.broadcasted_iota(jnp.int32, (C_PAD, 1), 0) < C
    msk = (seg == gids) & valid
    mf = msk.astype(jnp.float32)
    neg = jnp.float32(-1e30)
    mx = jnp.max(jnp.where(msk, logits, neg), axis=0, keepdims=True)
    mxc = jnp.sum(mf * mx, axis=1, keepdims=True)
    sh = logits - mxc
    ex = jnp.where(valid, jnp.exp(sh), 0.0)
    sg = jnp.sum(mf * ex, axis=0, keepdims=True)
    lse = jnp.log(jnp.maximum(sg, 1e-30))
    lsec = jnp.sum(mf * lse, axis=1, keepdims=True)
    out_ref[...] = sh - lsec


def _readout(hc, seg2d, wout, bout):
    return pl.pallas_call(
        _readout_body,
        in_specs=[
            pl.BlockSpec((C_PAD, EMB), lambda: (0, 0)),
            pl.BlockSpec((C_PAD, 1), lambda: (0, 0)),
            pl.BlockSpec((1, EMB), lambda: (0, 0)),
            pl.BlockSpec((1, 1), lambda: (0, 0)),
        ],
        out_specs=pl.BlockSpec((C_PAD, 1), lambda: (0, 0)),
        out_shape=jax.ShapeDtypeStruct((C_PAD, 1), jnp.float32),
    )(hc, seg2d, wout, bout)


def kernel(x, edge_index, edge_attr, candidate_idxs, batch,
           lin_in_w, lin_in_b, w1, b1, w2, b2, g1, be1, g2, be2,
           lin_out_w, lin_out_b):
    f32 = jnp.float32
    src = edge_index[0]
    dst = edge_index[1]
    pad = E_PAD - E
    dst_g = jnp.concatenate([dst, jnp.zeros((pad,), jnp.int32)])
    src_g = jnp.concatenate([src, jnp.zeros((pad,), jnp.int32)])
    dst_s = jnp.concatenate([dst, jnp.full((pad,), N, jnp.int32)])
    ea_p = jnp.concatenate([edge_attr, jnp.zeros((pad, ED), f32)], axis=0)
    cand_p = jnp.concatenate(
        [candidate_idxs, jnp.zeros((C_PAD - C,), jnp.int32)])

    w_in_t = lin_in_w.T
    b_in = lin_in_b[None, :]
    a_t = jnp.transpose(w1[:, :, :EMB], (0, 2, 1))
    b_t = jnp.transpose(w1[:, :, EMB:2 * EMB], (0, 2, 1))
    c_t = jnp.transpose(w1[:, :, 2 * EMB:], (0, 2, 1))
    w2_t = jnp.transpose(w2, (0, 2, 1))
    ef = jnp.float32(E)

    h, p, q = _node_first(x, w_in_t, b_in, a_t[0], b_t[0])
    for l in range(NLAYERS):
        gd, gs = _gather_pq(p, q, dst_g, src_g)
        m, s1s, s2s = _edge_pass1(gd, gs, ea_p, c_t[l], b1[l][None, :])
        mean1 = s1s[0] / ef
        var1 = s2s[0] / ef - mean1 * mean1
        sc1 = g1[l] * lax.rsqrt(var1 + EPS)
        sh1 = be1[l] - mean1 * sc1
        m2, u1s, u2s = _edge_pass2(m, sc1[None, :], sh1[None, :],
                                   w2_t[l], b2[l][None, :])
        mean2 = u1s[0] / ef
        var2 = u2s[0] / ef - mean2 * mean2
        sc2 = g2[l] * lax.rsqrt(var2 + EPS)
        sh2 = be2[l] - mean2 * sc2
        agg = _scatter_layer(m2, dst_s, sc2, sh2)
        a0 = agg[0, :N]
        a1 = agg[1, :N]
        if l < NLAYERS - 1:
            h, p, q = _node_update(h, a0, a1, a_t[l + 1], b_t[l + 1])
        else:
            h = _node_final(h, a0, a1)

    hc, seg = _readout_gather(h, cand_p, batch)
    out2d = _readout(hc, seg.reshape(C_PAD, 1),
                     lin_out_w, lin_out_b.reshape(1, 1))
    return out2d[:C, 0]


# speedup vs baseline: 1.9657x; 1.1465x over previous
import functools

import jax
import jax.numpy as jnp
from jax import lax
from jax.experimental import pallas as pl
from jax.experimental.pallas import tpu as pltpu
from jax.experimental.pallas import tpu_sc as plsc

N = 50000
E = 800000
C = 5000
G = 500
EMB = 32
ED = 4
IN_DIM = 2
NLAYERS = 4
EPS = 1e-5

NC = 2
NS = 16
NW = NC * NS

ECHUNK = 128
CHUNKS_PER_W = 196
E_PAD = NW * CHUNKS_PER_W * ECHUNK

NACC = 51200
ROWS_PER_SUB = NACC // NS

C_CHUNK = 80
C_PAD = NW * 2 * C_CHUNK

ETILE = 2048
EGRID = E_PAD // ETILE
NTILE = 2000
NGRID = N // NTILE

_MESH = plsc.VectorSubcoreMesh(core_axis_name="c", subcore_axis_name="s")
_SC_PARAMS = pltpu.CompilerParams(use_tc_tiling_on_sc=False,
                                  needs_layout_passes=False)


@functools.partial(
    pl.kernel,
    out_type=jax.ShapeDtypeStruct((E_PAD, EMB), jnp.float32),
    mesh=_MESH,
    compiler_params=_SC_PARAMS,
    scratch_types=[
        pltpu.VMEM((CHUNKS_PER_W, ECHUNK), jnp.int32),
        pltpu.VMEM((CHUNKS_PER_W, ECHUNK), jnp.int32),
        pltpu.VMEM((2, ECHUNK, EMB), jnp.float32),
        pltpu.VMEM((2, ECHUNK, EMB), jnp.float32),
        pltpu.SemaphoreType.DMA((2,)),
        pltpu.SemaphoreType.DMA((2,)),
        pltpu.SemaphoreType.DMA((2,)),
    ],
)
def _gather_pq(p_hbm, q_hbm, dst_hbm, src_hbm, g_hbm,
               idxd, idxs, bufp, bufq, psem, qsem, wsem):
    wid = lax.axis_index("s") * NC + lax.axis_index("c")
    pltpu.sync_copy(dst_hbm.at[wid], idxd)
    pltpu.sync_copy(src_hbm.at[wid], idxs)

    def start(j, b):
        pltpu.async_copy(p_hbm.at[idxd.at[j]], bufp.at[b], psem.at[b])
        pltpu.async_copy(q_hbm.at[idxs.at[j]], bufq.at[b], qsem.at[b])

    def out_slice(j):
        return g_hbm.at[pl.ds((wid * CHUNKS_PER_W + j) * ECHUNK, ECHUNK)]

    start(0, 0)

    def chunk(j, carry):
        b = j & 1
        pltpu.make_async_copy(p_hbm.at[idxd.at[j]], bufp.at[b], psem.at[b]).wait()
        pltpu.make_async_copy(q_hbm.at[idxs.at[j]], bufq.at[b], qsem.at[b]).wait()

        @pl.when(j + 1 < CHUNKS_PER_W)
        def _():
            @pl.when(j >= 1)
            def _():
                pltpu.make_async_copy(bufp.at[1 - b], out_slice(j - 1),
                                      wsem.at[1 - b]).wait()
            start(j + 1, 1 - b)

        def add4(r, rc):
            for k in range(4):
                row = r * 4 + k
                for half in range(2):
                    sl = pl.ds(half * 16, 16)
                    bufp[b, row, sl] = bufp[b, row, sl] + bufq[b, row, sl]
            return rc

        lax.fori_loop(0, ECHUNK // 4, add4, 0)
        pltpu.async_copy(bufp.at[b], out_slice(j), wsem.at[b])
        return carry

    lax.fori_loop(0, CHUNKS_PER_W, chunk, 0)
    pltpu.make_async_copy(bufp.at[0], out_slice(CHUNKS_PER_W - 2), wsem.at[0]).wait()
    pltpu.make_async_copy(bufp.at[1], out_slice(CHUNKS_PER_W - 1), wsem.at[1]).wait()


@functools.partial(
    pl.kernel,
    out_type=jax.ShapeDtypeStruct((NC, NACC, EMB), jnp.float32),
    mesh=_MESH,
    compiler_params=_SC_PARAMS,
    scratch_types=[
        pltpu.VMEM((2, ECHUNK, EMB), jnp.float32),
        pltpu.VMEM((2, ECHUNK, EMB), jnp.float32),
        pltpu.VMEM((2, ECHUNK), jnp.int32),
        pltpu.VMEM((EMB,), jnp.float32),
        pltpu.VMEM((EMB,), jnp.float32),
        pltpu.VMEM_SHARED((NACC, EMB), jnp.float32),
        pltpu.SemaphoreType.DMA((2,)),
        pltpu.SemaphoreType.DMA((2,)),
        pltpu.SemaphoreType.DMA((2,)),
    ],
)
def _scatter_layer(m2_hbm, dst_hbm, s_hbm, t_hbm, out_hbm,
                   mbuf, obuf, idxb, s2v, t2v, acc, lsem, isem, ssem):
    cid = lax.axis_index("c")
    sid = lax.axis_index("s")
    wid = sid * NC + cid

    zero = jnp.zeros((16,), jnp.float32)

    def zrow(r, carry):
        obuf[0, r, pl.ds(0, 16)] = zero
        obuf[0, r, pl.ds(16, 16)] = zero
        return carry

    lax.fori_loop(0, ECHUNK, zrow, 0)

    def zstripe(j, carry):
        pltpu.sync_copy(obuf.at[0],
                        acc.at[pl.ds(sid * ROWS_PER_SUB + j * ECHUNK, ECHUNK)])
        return carry

    lax.fori_loop(0, ROWS_PER_SUB // ECHUNK, zstripe, 0)

    pltpu.sync_copy(s_hbm, s2v)
    pltpu.sync_copy(t_hbm, t2v)
    sa = s2v[pl.ds(0, 16)]
    sb = s2v[pl.ds(16, 16)]
    ta = t2v[pl.ds(0, 16)]
    tb = t2v[pl.ds(16, 16)]

    plsc.subcore_barrier()

    def in_slice(j):
        return m2_hbm.at[pl.ds((wid * CHUNKS_PER_W + j) * ECHUNK, ECHUNK)]

    def idx_slice(j):
        return dst_hbm.at[wid, j]

    def start(j, b):
        pltpu.async_copy(in_slice(j), mbuf.at[b], lsem.at[b])
        pltpu.async_copy(idx_slice(j), idxb.at[b], isem.at[b])

    start(0, 0)

    def chunk(j, carry):
        b = j & 1
        pltpu.make_async_copy(in_slice(j), mbuf.at[b], lsem.at[b]).wait()
        pltpu.make_async_copy(idx_slice(j), idxb.at[b], isem.at[b]).wait()

        def row4(r, rc):
            for k in range(4):
                row = r * 4 + k
                s0 = pl.ds(0, 16)
                s1 = pl.ds(16, 16)
                obuf[b, row, s0] = jnp.maximum(mbuf[b, row, s0] * sa + ta, 0.0)
                obuf[b, row, s1] = jnp.maximum(mbuf[b, row, s1] * sb + tb, 0.0)
            return rc

        lax.fori_loop(0, ECHUNK // 4, row4, 0)
        pltpu.async_copy(obuf.at[b], acc.at[idxb.at[b]], ssem.at[b], add=True)

        @pl.when(j >= 1)
        def _():
            pltpu.make_async_copy(obuf.at[1 - b], acc.at[idxb.at[1 - b]],
                                  ssem.at[1 - b]).wait()

        @pl.when(j + 1 < CHUNKS_PER_W)
        def _():
            start(j + 1, 1 - b)

        return carry

    lax.fori_loop(0, CHUNKS_PER_W, chunk, 0)
    pltpu.make_async_copy(obuf.at[1], acc.at[idxb.at[1]], ssem.at[1]).wait()

    plsc.subcore_barrier()
    pltpu.sync_copy(acc.at[pl.ds(sid * ROWS_PER_SUB, ROWS_PER_SUB)],
                    out_hbm.at[cid, pl.ds(sid * ROWS_PER_SUB, ROWS_PER_SUB)])


@functools.partial(
    pl.kernel,
    out_type=(
        jax.ShapeDtypeStruct((C_PAD, EMB), jnp.float32),
        jax.ShapeDtypeStruct((C_PAD,), jnp.int32),
    ),
    mesh=_MESH,
    compiler_params=_SC_PARAMS,
    scratch_types=[
        pltpu.VMEM((C_CHUNK,), jnp.int32),
        pltpu.VMEM((C_CHUNK, EMB), jnp.float32),
        pltpu.VMEM((C_CHUNK,), jnp.int32),
        pltpu.VMEM((N,), jnp.int32),
        pltpu.SemaphoreType.DMA,
    ],
)
def _readout_gather(h_hbm, cand_hbm, batch_hbm, hc_hbm, seg_hbm,
                    idxc, bufh, segbuf, batchv, sem):
    wid = lax.axis_index("s") * NC + lax.axis_index("c")
    pltpu.sync_copy(batch_hbm, batchv)
    for k in range(2):
        base = wid * (2 * C_CHUNK) + k * C_CHUNK
        pltpu.sync_copy(cand_hbm.at[pl.ds(base, C_CHUNK)], idxc)
        pltpu.async_copy(h_hbm.at[idxc], bufh, sem).wait()
        for t in range(C_CHUNK // 16):
            iv = idxc[pl.ds(t * 16, 16)]
            segbuf[pl.ds(t * 16, 16)] = plsc.load_gather(batchv, [iv])
        pltpu.sync_copy(bufh, hc_hbm.at[pl.ds(base, C_CHUNK)])
        pltpu.sync_copy(segbuf, seg_hbm.at[pl.ds(base, C_CHUNK)])


def _node_first_body(x_ref, w_ref, b_ref, at_ref, bt_ref, h_ref, p_ref, q_ref):
    h = jnp.dot(x_ref[...], w_ref[...], preferred_element_type=jnp.float32) + b_ref[...]
    h_ref[...] = h
    p_ref[...] = jnp.dot(h, at_ref[...], preferred_element_type=jnp.float32)
    q_ref[...] = jnp.dot(h, bt_ref[...], preferred_element_type=jnp.float32)


def _node_first(x, w_in_t, b_in, a_t, b_t):
    return pl.pallas_call(
        _node_first_body,
        grid=(NGRID,),
        in_specs=[
            pl.BlockSpec((NTILE, IN_DIM), lambda i: (i, 0)),
            pl.BlockSpec((IN_DIM, EMB), lambda i: (0, 0)),
            pl.BlockSpec((1, EMB), lambda i: (0, 0)),
            pl.BlockSpec((EMB, EMB), lambda i: (0, 0)),
            pl.BlockSpec((EMB, EMB), lambda i: (0, 0)),
        ],
        out_specs=[
            pl.BlockSpec((NTILE, EMB), lambda i: (i, 0)),
            pl.BlockSpec((NTILE, EMB), lambda i: (i, 0)),
            pl.BlockSpec((NTILE, EMB), lambda i: (i, 0)),
        ],
        out_shape=[
            jax.ShapeDtypeStruct((N, EMB), jnp.float32),
            jax.ShapeDtypeStruct((N, EMB), jnp.float32),
            jax.ShapeDtypeStruct((N, EMB), jnp.float32),
        ],
    )(x, w_in_t, b_in, a_t, b_t)


def _node_update_body(h_ref, a0_ref, a1_ref, at_ref, bt_ref,
                      ho_ref, p_ref, q_ref):
    h = h_ref[...] + a0_ref[...] + a1_ref[...]
    ho_ref[...] = h
    p_ref[...] = jnp.dot(h, at_ref[...], preferred_element_type=jnp.float32)
    q_ref[...] = jnp.dot(h, bt_ref[...], preferred_element_type=jnp.float32)


def _node_update(h, a0, a1, a_t, b_t):
    return pl.pallas_call(
        _node_update_body,
        grid=(NGRID,),
        in_specs=[
            pl.BlockSpec((NTILE, EMB), lambda i: (i, 0)),
            pl.BlockSpec((NTILE, EMB), lambda i: (i, 0)),
            pl.BlockSpec((NTILE, EMB), lambda i: (i, 0)),
            pl.BlockSpec((EMB, EMB), lambda i: (0, 0)),
            pl.BlockSpec((EMB, EMB), lambda i: (0, 0)),
        ],
        out_specs=[
            pl.BlockSpec((NTILE, EMB), lambda i: (i, 0)),
            pl.BlockSpec((NTILE, EMB), lambda i: (i, 0)),
            pl.BlockSpec((NTILE, EMB), lambda i: (i, 0)),
        ],
        out_shape=[
            jax.ShapeDtypeStruct((N, EMB), jnp.float32),
            jax.ShapeDtypeStruct((N, EMB), jnp.float32),
            jax.ShapeDtypeStruct((N, EMB), jnp.float32),
        ],
    )(h, a0, a1, a_t, b_t)


def _node_final_body(h_ref, a0_ref, a1_ref, ho_ref):
    ho_ref[...] = h_ref[...] + a0_ref[...] + a1_ref[...]


def _node_final(h, a0, a1):
    return pl.pallas_call(
        _node_final_body,
        grid=(NGRID,),
        in_specs=[
            pl.BlockSpec((NTILE, EMB), lambda i: (i, 0)),
            pl.BlockSpec((NTILE, EMB), lambda i: (i, 0)),
            pl.BlockSpec((NTILE, EMB), lambda i: (i, 0)),
        ],
        out_specs=pl.BlockSpec((NTILE, EMB), lambda i: (i, 0)),
        out_shape=jax.ShapeDtypeStruct((N, EMB), jnp.float32),
    )(h, a0, a1)


def _edge1_body(g_ref, ea_ref, ct_ref, b1_ref, m_ref, s1_ref, s2_ref):
    i = pl.program_id(0)
    m = (g_ref[...]
         + jnp.dot(ea_ref[...], ct_ref[...], preferred_element_type=jnp.float32)
         + b1_ref[...])
    m_ref[...] = m
    rows = lax.broadcasted_iota(jnp.int32, (ETILE, EMB), 0) + i * ETILE
    mask = rows < E
    p1 = jnp.sum(jnp.where(mask, m, 0.0), axis=0)
    p2 = jnp.sum(jnp.where(mask, m * m, 0.0), axis=0)
    rowsel = lax.broadcasted_iota(jnp.int32, (8, EMB), 0) == 0
    c1 = jnp.where(rowsel, p1[None, :], 0.0)
    c2 = jnp.where(rowsel, p2[None, :], 0.0)

    @pl.when(i == 0)
    def _():
        s1_ref[...] = c1
        s2_ref[...] = c2

    @pl.when(i > 0)
    def _():
        s1_ref[...] += c1
        s2_ref[...] += c2


def _edge_pass1(g, ea, c_t, b1r):
    return pl.pallas_call(
        _edge1_body,
        grid=(EGRID,),
        in_specs=[
            pl.BlockSpec((ETILE, EMB), lambda i: (i, 0)),
            pl.BlockSpec((ETILE, ED), lambda i: (i, 0)),
            pl.BlockSpec((ED, EMB), lambda i: (0, 0)),
            pl.BlockSpec((1, EMB), lambda i: (0, 0)),
        ],
        out_specs=[
            pl.BlockSpec((ETILE, EMB), lambda i: (i, 0)),
            pl.BlockSpec((8, EMB), lambda i: (0, 0)),
            pl.BlockSpec((8, EMB), lambda i: (0, 0)),
        ],
        out_shape=[
            jax.ShapeDtypeStruct((E_PAD, EMB), jnp.float32),
            jax.ShapeDtypeStruct((8, EMB), jnp.float32),
            jax.ShapeDtypeStruct((8, EMB), jnp.float32),
        ],
    )(g, ea, c_t, b1r)


def _edge2_body(m_ref, s1_ref, t1_ref, w2t_ref, b2_ref, m2_ref, u1_ref, u2_ref):
    i = pl.program_id(0)
    a = jnp.maximum(m_ref[...] * s1_ref[...] + t1_ref[...], 0.0)
    m2 = jnp.dot(a, w2t_ref[...], preferred_element_type=jnp.float32) + b2_ref[...]
    m2_ref[...] = m2
    rows = lax.broadcasted_iota(jnp.int32, (ETILE, EMB), 0) + i * ETILE
    mask = rows < E
    p1 = jnp.sum(jnp.where(mask, m2, 0.0), axis=0)
    p2 = jnp.sum(jnp.where(mask, m2 * m2, 0.0), axis=0)
    rowsel = lax.broadcasted_iota(jnp.int32, (8, EMB), 0) == 0
    c1 = jnp.where(rowsel, p1[None, :], 0.0)
    c2 = jnp.where(rowsel, p2[None, :], 0.0)

    @pl.when(i == 0)
    def _():
        u1_ref[...] = c1
        u2_ref[...] = c2

    @pl.when(i > 0)
    def _():
        u1_ref[...] += c1
        u2_ref[...] += c2


def _edge_pass2(m, s1r, t1r, w2t, b2r):
    return pl.pallas_call(
        _edge2_body,
        grid=(EGRID,),
        in_specs=[
            pl.BlockSpec((ETILE, EMB), lambda i: (i, 0)),
            pl.BlockSpec((1, EMB), lambda i: (0, 0)),
            pl.BlockSpec((1, EMB), lambda i: (0, 0)),
            pl.BlockSpec((EMB, EMB), lambda i: (0, 0)),
            pl.BlockSpec((1, EMB), lambda i: (0, 0)),
        ],
        out_specs=[
            pl.BlockSpec((ETILE, EMB), lambda i: (i, 0)),
            pl.BlockSpec((8, EMB), lambda i: (0, 0)),
            pl.BlockSpec((8, EMB), lambda i: (0, 0)),
        ],
        out_shape=[
            jax.ShapeDtypeStruct((E_PAD, EMB), jnp.float32),
            jax.ShapeDtypeStruct((8, EMB), jnp.float32),
            jax.ShapeDtypeStruct((8, EMB), jnp.float32),
        ],
    )(m, s1r, t1r, w2t, b2r)


def _readout_body(hc_ref, seg_ref, wout_ref, bout_ref, out_ref):
    hc = hc_ref[...]
    logits = jnp.sum(hc * wout_ref[...], axis=1, keepdims=True) + bout_ref[...]
    seg = seg_ref[...]
    gids = lax.broadcasted_iota(jnp.int32, (C_PAD, G), 1)
    valid = lax.broadcasted_iota(jnp.int32, (C_PAD, 1), 0) < C
    msk = (seg == gids) & valid
    mf = msk.astype(jnp.float32)
    neg = jnp.float32(-1e30)
    mx = jnp.max(jnp.where(msk, logits, neg), axis=0, keepdims=True)
    mxc = jnp.sum(mf * mx, axis=1, keepdims=True)
    sh = logits - mxc
    ex = jnp.where(valid, jnp.exp(sh), 0.0)
    sg = jnp.sum(mf * ex, axis=0, keepdims=True)
    lse = jnp.log(jnp.maximum(sg, 1e-30))
    lsec = jnp.sum(mf * lse, axis=1, keepdims=True)
    out_ref[...] = sh - lsec


def _readout(hc, seg2d, wout, bout):
    return pl.pallas_call(
        _readout_body,
        in_specs=[
            pl.BlockSpec((C_PAD, EMB), lambda: (0, 0)),
            pl.BlockSpec((C_PAD, 1), lambda: (0, 0)),
            pl.BlockSpec((1, EMB), lambda: (0, 0)),
            pl.BlockSpec((1, 1), lambda: (0, 0)),
        ],
        out_specs=pl.BlockSpec((C_PAD, 1), lambda: (0, 0)),
        out_shape=jax.ShapeDtypeStruct((C_PAD, 1), jnp.float32),
    )(hc, seg2d, wout, bout)


def kernel(x, edge_index, edge_attr, candidate_idxs, batch,
           lin_in_w, lin_in_b, w1, b1, w2, b2, g1, be1, g2, be2,
           lin_out_w, lin_out_b):
    f32 = jnp.float32
    src = edge_index[0]
    dst = edge_index[1]
    pad = E_PAD - E
    dst_g = jnp.concatenate([dst, jnp.zeros((pad,), jnp.int32)]).reshape(
        NW, CHUNKS_PER_W, ECHUNK)
    src_g = jnp.concatenate([src, jnp.zeros((pad,), jnp.int32)]).reshape(
        NW, CHUNKS_PER_W, ECHUNK)
    dst_s = jnp.concatenate([dst, jnp.full((pad,), N, jnp.int32)]).reshape(
        NW, CHUNKS_PER_W, ECHUNK)
    ea_p = jnp.concatenate([edge_attr, jnp.zeros((pad, ED), f32)], axis=0)
    cand_p = jnp.concatenate(
        [candidate_idxs, jnp.zeros((C_PAD - C,), jnp.int32)])

    w_in_t = lin_in_w.T
    b_in = lin_in_b[None, :]
    a_t = jnp.transpose(w1[:, :, :EMB], (0, 2, 1))
    b_t = jnp.transpose(w1[:, :, EMB:2 * EMB], (0, 2, 1))
    c_t = jnp.transpose(w1[:, :, 2 * EMB:], (0, 2, 1))
    w2_t = jnp.transpose(w2, (0, 2, 1))
    ef = jnp.float32(E)

    h, p, q = _node_first(x, w_in_t, b_in, a_t[0], b_t[0])
    for l in range(NLAYERS):
        g = _gather_pq(p, q, dst_g, src_g)
        m, s1s, s2s = _edge_pass1(g, ea_p, c_t[l], b1[l][None, :])
        mean1 = s1s[0] / ef
        var1 = s2s[0] / ef - mean1 * mean1
        sc1 = g1[l] * lax.rsqrt(var1 + EPS)
        sh1 = be1[l] - mean1 * sc1
        m2, u1s, u2s = _edge_pass2(m, sc1[None, :], sh1[None, :],
                                   w2_t[l], b2[l][None, :])
        mean2 = u1s[0] / ef
        var2 = u2s[0] / ef - mean2 * mean2
        sc2 = g2[l] * lax.rsqrt(var2 + EPS)
        sh2 = be2[l] - mean2 * sc2
        agg = _scatter_layer(m2, dst_s, sc2, sh2)
        a0 = agg[0, :N]
        a1 = agg[1, :N]
        if l < NLAYERS - 1:
            h, p, q = _node_update(h, a0, a1, a_t[l + 1], b_t[l + 1])
        else:
            h = _node_final(h, a0, a1)

    hc, seg = _readout_gather(h, cand_p, batch)
    out2d = _readout(hc, seg.reshape(C_PAD, 1),
                     lin_out_w, lin_out_b.reshape(1, 1))
    return out2d[:C, 0]
